# baseline (speedup 1.0000x reference)
import functools

import jax
import jax.numpy as jnp
from jax import lax
from jax.experimental import pallas as pl
from jax.experimental.pallas import tpu as pltpu
from jax.experimental.pallas import tpu_sc as plsc

N = 10000
E = 320000
DIM = 128
G = 64
L = 3

NC = 2
NS = 16
NW = NC * NS
EPT = E // NW
K = 80
CPG = 8
CH = 128
NGRP = CH // CPG
D = 4
PADN = 10240
PADROW = 10200
RPT = PADN // NS
ZR = 64

BLK = 1000
NB = N // BLK


def _sc_agg(h, e4):
    mesh = plsc.VectorSubcoreMesh(core_axis_name="c", subcore_axis_name="s")

    @functools.partial(
        pl.kernel,
        mesh=mesh,
        out_type=jax.ShapeDtypeStruct((NC, PADN, DIM), jnp.float32),
        scratch_types=[
            pltpu.VMEM((2, CPG, K), jnp.int32),
            pltpu.VMEM((2, CPG, K), jnp.int32),
            pltpu.VMEM((K, DIM), jnp.float32),
            pltpu.VMEM((K, DIM), jnp.float32),
            pltpu.VMEM((K, DIM), jnp.float32),
            pltpu.VMEM((K, DIM), jnp.float32),
            pltpu.VMEM_SHARED((PADN, DIM), jnp.float32),
            pltpu.SemaphoreType.DMA,
            pltpu.SemaphoreType.DMA,
            pltpu.SemaphoreType.DMA,
            pltpu.SemaphoreType.DMA,
            pltpu.SemaphoreType.DMA,
        ],
    )
    def agg(h_hbm, e_hbm, out_hbm, idx_s, idx_d,
            rows0, rows1, rows2, rows3, acc, sem0, sem1, sem2, sem3, semi):
        c = lax.axis_index("c")
        s = lax.axis_index("s")
        wid = s * NC + c

        pltpu.async_copy(e_hbm.at[0, wid, pl.ds(0, CPG)], idx_s.at[0], semi)
        pltpu.async_copy(e_hbm.at[1, wid, pl.ds(0, CPG)], idx_d.at[0], semi)
        pltpu.async_copy(e_hbm.at[0, wid, pl.ds(CPG, CPG)], idx_s.at[1], semi)
        pltpu.async_copy(e_hbm.at[1, wid, pl.ds(CPG, CPG)], idx_d.at[1], semi)

        def zb(i, carry):
            for j in range(8):
                rows0[i, pl.ds(j * 16, 16)] = jnp.zeros((16,), jnp.float32)
            return carry
        lax.fori_loop(0, ZR, zb, 0)
        row0 = s * RPT
        zstg = rows0.at[pl.ds(0, ZR)]

        def zacc(j, carry):
            pltpu.async_copy(zstg, acc.at[pl.ds(row0 + j * ZR, ZR)], sem0)
            return carry
        lax.fori_loop(0, RPT // ZR, zacc, 0)

        def zdrain(j, carry):
            pltpu.make_async_copy(zstg, acc.at[pl.ds(row0, ZR)], sem0).wait()
            return carry
        lax.fori_loop(0, RPT // ZR, zdrain, 0)
        plsc.subcore_barrier()

        rows = (rows0, rows1, rows2, rows3)
        sems = (sem0, sem1, sem2, sem3)

        pltpu.make_async_copy(e_hbm.at[0, wid, pl.ds(0, CPG)],
                              idx_s.at[0], semi).wait()
        pltpu.make_async_copy(e_hbm.at[1, wid, pl.ds(0, CPG)],
                              idx_d.at[0], semi).wait()
        for b in range(D - 1):
            pltpu.async_copy(h_hbm.at[idx_s.at[0, b]], rows[b], sems[b])

        def group(g, carry):
            p = g % 2
            for b in range(CPG):
                cur = b % D
                ahead = b + D - 1
                nb = ahead % D
                if ahead < CPG:
                    pltpu.async_copy(h_hbm.at[idx_s.at[p, ahead]],
                                     rows[nb], sems[nb])
                else:
                    if ahead == CPG:
                        @pl.when(g < NGRP - 1)
                        def _():
                            pltpu.make_async_copy(
                                e_hbm.at[0, wid, pl.ds(0, CPG)],
                                idx_s.at[0], semi).wait()
                            pltpu.make_async_copy(
                                e_hbm.at[1, wid, pl.ds(0, CPG)],
                                idx_d.at[0], semi).wait()

                    @pl.when(g < NGRP - 1)
                    def _():
                        pltpu.async_copy(
                            h_hbm.at[idx_s.at[1 - p, ahead - CPG]],
                            rows[nb], sems[nb])
                pltpu.make_async_copy(h_hbm.at[idx_s.at[p, b]],
                                      rows[cur], sems[cur]).wait()
                pltpu.sync_copy(rows[cur], acc.at[idx_d.at[p, b]], add=True)
            @pl.when(g < NGRP - 2)
            def _():
                pltpu.async_copy(
                    e_hbm.at[0, wid, pl.ds((g + 2) * CPG, CPG)],
                    idx_s.at[p], semi)
                pltpu.async_copy(
                    e_hbm.at[1, wid, pl.ds((g + 2) * CPG, CPG)],
                    idx_d.at[p], semi)
            return carry
        lax.fori_loop(0, NGRP, group, 0)

        plsc.subcore_barrier()
        pltpu.sync_copy(acc.at[pl.ds(row0, RPT)],
                        out_hbm.at[c, pl.ds(row0, RPT)])

    return agg(h, e4)


def _tc_mlp(h, p, oh, W1, b1, W2, b2, write_z=True):
    def body(h_ref, p_ref, oh_ref, w1_ref, b1_ref, w2_ref, b2_ref,
             *out_refs):
        if write_z:
            z_ref, stats_ref, gsum_ref, cnt_ref = out_refs
        else:
            stats_ref, gsum_ref, cnt_ref = out_refs
        b = pl.program_id(0)
        u = h_ref[...] + p_ref[0] + p_ref[1]
        a = jnp.maximum(
            jnp.dot(u, w1_ref[...], preferred_element_type=jnp.float32)
            + b1_ref[...], 0.0)
        z = jnp.maximum(
            jnp.dot(a, w2_ref[...], preferred_element_type=jnp.float32)
            + b2_ref[...], 0.0)
        if write_z:
            z_ref[...] = z

        ohb = oh_ref[...]
        gs = lax.dot_general(ohb, z, (((0,), (0,)), ((), ())),
                             preferred_element_type=jnp.float32)
        cn = lax.dot_general(ohb, jnp.ones((BLK, DIM), jnp.float32),
                             (((0,), (0,)), ((), ())),
                             preferred_element_type=jnp.float32)
        s0 = jnp.sum(z, axis=0, keepdims=True)
        s1 = jnp.sum(z * z, axis=0, keepdims=True)
        st = jnp.concatenate(
            [s0, s1, jnp.zeros((6, DIM), jnp.float32)], axis=0)

        init = b == 0
        stats_ref[...] = jnp.where(init, st, stats_ref[...] + st)
        gsum_ref[...] = jnp.where(init, gs, gsum_ref[...] + gs)
        cnt_ref[...] = jnp.where(init, cn, cnt_ref[...] + cn)

    zspec = [pl.BlockSpec((BLK, DIM), lambda b: (b, 0))] if write_z else []
    zshape = [jax.ShapeDtypeStruct((N, DIM), jnp.float32)] if write_z else []
    out = pl.pallas_call(
        body,
        grid=(NB,),
        in_specs=[
            pl.BlockSpec((BLK, DIM), lambda b: (b, 0)),
            pl.BlockSpec((NC, BLK, DIM), lambda b: (0, b, 0)),
            pl.BlockSpec((BLK, G), lambda b: (b, 0)),
            pl.BlockSpec((DIM, DIM), lambda b: (0, 0)),
            pl.BlockSpec((1, DIM), lambda b: (0, 0)),
            pl.BlockSpec((DIM, DIM), lambda b: (0, 0)),
            pl.BlockSpec((1, DIM), lambda b: (0, 0)),
        ],
        out_specs=zspec + [
            pl.BlockSpec((8, DIM), lambda b: (0, 0)),
            pl.BlockSpec((G, DIM), lambda b: (0, 0)),
            pl.BlockSpec((G, DIM), lambda b: (0, 0)),
        ],
        out_shape=zshape + [
            jax.ShapeDtypeStruct((8, DIM), jnp.float32),
            jax.ShapeDtypeStruct((G, DIM), jnp.float32),
            jax.ShapeDtypeStruct((G, DIM), jnp.float32),
        ],
    )(h, p, oh, W1, b1, W2, b2)
    if not write_z:
        return (None,) + tuple(out)
    return out


def _tc_norm(z, stats, gamma, beta):
    def body(z_ref, st_ref, g_ref, be_ref, h_ref):
        mean = st_ref[0:1, :] * (1.0 / N)
        msq = st_ref[1:2, :] * (1.0 / N)
        var = msq - mean * mean
        rstd = lax.rsqrt(var + 1e-5)
        h_ref[...] = (z_ref[...] - mean) * (rstd * g_ref[...]) + be_ref[...]

    return pl.pallas_call(
        body,
        grid=(NB,),
        in_specs=[
            pl.BlockSpec((BLK, DIM), lambda b: (b, 0)),
            pl.BlockSpec((8, DIM), lambda b: (0, 0)),
            pl.BlockSpec((1, DIM), lambda b: (0, 0)),
            pl.BlockSpec((1, DIM), lambda b: (0, 0)),
        ],
        out_specs=pl.BlockSpec((BLK, DIM), lambda b: (b, 0)),
        out_shape=jax.ShapeDtypeStruct((N, DIM), jnp.float32),
    )(z, stats, gamma, beta)


def _tc_combine(gsums, cnt, statss, gammas, betas):
    def body(g0, g1, g2, cnt_ref, s0, s1, s2, ga0, ga1, ga2,
             be0, be1, be2, out_ref):
        gr = (g0, g1, g2)
        sr = (s0, s1, s2)
        gar = (ga0, ga1, ga2)
        ber = (be0, be1, be2)
        c = cnt_ref[...]
        for i in range(L):
            mean = sr[i][0:1, :] * (1.0 / N)
            msq = sr[i][1:2, :] * (1.0 / N)
            var = msq - mean * mean
            rstd = lax.rsqrt(var + 1e-5)
            pool = ((gr[i][...] - c * mean) * (rstd * gar[i][...])
                    + c * ber[i][...])
            out_ref[:, DIM * i:DIM * (i + 1)] = pool

    full = lambda shp: pl.BlockSpec(shp, lambda: tuple(0 for _ in shp))
    return pl.pallas_call(
        body,
        in_specs=[full((G, DIM))] * 3 + [full((G, DIM))] + [full((8, DIM))] * 3
                 + [full((1, DIM))] * 6,
        out_specs=full((G, L * DIM)),
        out_shape=jax.ShapeDtypeStruct((G, L * DIM), jnp.float32),
    )(*gsums, cnt, *statss, *gammas, *betas)


def kernel(x, edge_index, batch,
           W1_0, b1_0, W2_0, b2_0, gamma_0, beta_0,
           W1_1, b1_1, W2_1, b2_1, gamma_1, beta_1,
           W1_2, b1_2, W2_2, b2_2, gamma_2, beta_2):
    npad = CH - EPT // K
    er = edge_index.reshape(2, NW, EPT // K, K)
    dmy = jnp.stack([jnp.zeros((K,), jnp.int32),
                     jnp.full((K,), PADROW, jnp.int32)])
    dmy = jnp.broadcast_to(dmy[:, None, None, :], (2, NW, npad, K))
    e4 = jnp.concatenate([er, dmy], axis=2)
    oh = (batch[:, None] == jnp.arange(G, dtype=jnp.int32)[None, :])
    oh = oh.astype(jnp.float32)

    Ws = [(W1_0, b1_0, W2_0, b2_0, gamma_0, beta_0),
          (W1_1, b1_1, W2_1, b2_1, gamma_1, beta_1),
          (W1_2, b1_2, W2_2, b2_2, gamma_2, beta_2)]

    h = x
    gsums, statss, gammas, betas = [], [], [], []
    cnt = None
    for i in range(L):
        W1, b1, W2, b2, ga, be = Ws[i]
        b1r = b1.reshape(1, DIM)
        b2r = b2.reshape(1, DIM)
        gar = ga.reshape(1, DIM)
        ber = be.reshape(1, DIM)
        p = _sc_agg(h, e4)
        z, stats, gsum, cnt = _tc_mlp(h, p, oh, W1, b1r, W2, b2r,
                                      write_z=i < L - 1)
        gsums.append(gsum)
        statss.append(stats)
        gammas.append(gar)
        betas.append(ber)
        if i < L - 1:
            h = _tc_norm(z, stats, gar, ber)
    return _tc_combine(gsums, cnt, statss, gammas, betas)


# speedup vs baseline: 3.0877x; 3.0877x over previous
import functools

import jax
import jax.numpy as jnp
from jax import lax
from jax.experimental import pallas as pl
from jax.experimental.pallas import tpu as pltpu
from jax.experimental.pallas import tpu_sc as plsc

N = 10000
E = 320000
DIM = 128
G = 64
L = 3

NC = 2
NS = 16
NW = NC * NS
EPT = E // NW
K = 125
CPG = 8
CH = 80
NGRP = CH // CPG
PADN = 10240
RPT = PADN // NS
ZR = 128

BLK = 1000
NB = N // BLK


def _sc_agg(h, e4):
    mesh = plsc.VectorSubcoreMesh(core_axis_name="c", subcore_axis_name="s")

    @functools.partial(
        pl.kernel,
        mesh=mesh,
        out_type=jax.ShapeDtypeStruct((NC, PADN, DIM), jnp.float32),
        scratch_types=[
            pltpu.VMEM((2, CPG, K), jnp.int32),
            pltpu.VMEM((2, CPG, K), jnp.int32),
            pltpu.VMEM((ZR, DIM), jnp.float32),
            pltpu.VMEM((ZR, DIM), jnp.float32),
            pltpu.VMEM_SHARED((PADN, DIM), jnp.float32),
            pltpu.SemaphoreType.DMA,
            pltpu.SemaphoreType.DMA,
            pltpu.SemaphoreType.DMA,
        ],
    )
    def agg(h_hbm, e_hbm, out_hbm,
            idx_s, idx_d, rows0, rows1, acc, sem0, sem1, semi):
        c = lax.axis_index("c")
        s = lax.axis_index("s")
        wid = s * NC + c

        pltpu.async_copy(e_hbm.at[0, wid, pl.ds(0, CPG)], idx_s.at[0], semi)
        pltpu.async_copy(e_hbm.at[1, wid, pl.ds(0, CPG)], idx_d.at[0], semi)
        pltpu.async_copy(e_hbm.at[0, wid, pl.ds(CPG, CPG)], idx_s.at[1], semi)
        pltpu.async_copy(e_hbm.at[1, wid, pl.ds(CPG, CPG)], idx_d.at[1], semi)

        def zb(i, carry):
            for j in range(8):
                rows0[i, pl.ds(j * 16, 16)] = jnp.zeros((16,), jnp.float32)
            return carry
        lax.fori_loop(0, ZR, zb, 0)
        row0 = s * RPT

        def zacc(j, carry):
            pltpu.async_copy(rows0, acc.at[pl.ds(row0 + j * ZR, ZR)], sem0)
            return carry
        lax.fori_loop(0, RPT // ZR, zacc, 0)

        def zdrain(j, carry):
            pltpu.make_async_copy(rows0, acc.at[pl.ds(row0, ZR)],
                                  sem0).wait()
            return carry
        lax.fori_loop(0, RPT // ZR, zdrain, 0)
        plsc.subcore_barrier()

        rows = (rows0.at[pl.ds(0, K)], rows1.at[pl.ds(0, K)])
        sems = (sem0, sem1)

        pltpu.make_async_copy(e_hbm.at[0, wid, pl.ds(0, CPG)],
                              idx_s.at[0], semi).wait()
        pltpu.make_async_copy(e_hbm.at[1, wid, pl.ds(0, CPG)],
                              idx_d.at[0], semi).wait()
        pltpu.async_copy(h_hbm.at[idx_s.at[0, 0]], rows[0], sems[0])

        def group(g, carry):
            p = g % 2
            for b in range(CPG):
                cur = b % 2
                nxt = 1 - cur
                if b < CPG - 1:
                    pltpu.async_copy(h_hbm.at[idx_s.at[p, b + 1]],
                                     rows[nxt], sems[nxt])
                else:
                    @pl.when(g < NGRP - 1)
                    def _():
                        pltpu.make_async_copy(
                            e_hbm.at[0, wid, pl.ds(0, CPG)],
                            idx_s.at[0], semi).wait()
                        pltpu.make_async_copy(
                            e_hbm.at[1, wid, pl.ds(0, CPG)],
                            idx_d.at[0], semi).wait()
                        pltpu.async_copy(h_hbm.at[idx_s.at[1 - p, 0]],
                                         rows[nxt], sems[nxt])
                pltpu.make_async_copy(h_hbm.at[idx_s.at[p, b]],
                                      rows[cur], sems[cur]).wait()
                pltpu.sync_copy(rows[cur], acc.at[idx_d.at[p, b]], add=True)
            @pl.when(g < NGRP - 2)
            def _():
                pltpu.async_copy(
                    e_hbm.at[0, wid, pl.ds((g + 2) * CPG, CPG)],
                    idx_s.at[p], semi)
                pltpu.async_copy(
                    e_hbm.at[1, wid, pl.ds((g + 2) * CPG, CPG)],
                    idx_d.at[p], semi)
            return carry
        lax.fori_loop(0, NGRP, group, 0)

        plsc.subcore_barrier()
        pltpu.sync_copy(acc.at[pl.ds(row0, RPT)],
                        out_hbm.at[c, pl.ds(row0, RPT)])

    return agg(h, e4)


def _tc_mlp(h, p, oh, W1, b1, W2, b2, write_z=True):
    def body(h_ref, p_ref, oh_ref, w1_ref, b1_ref, w2_ref, b2_ref,
             *out_refs):
        if write_z:
            z_ref, stats_ref, gsum_ref, cnt_ref = out_refs
        else:
            stats_ref, gsum_ref, cnt_ref = out_refs
        b = pl.program_id(0)
        u = h_ref[...] + p_ref[0] + p_ref[1]
        a = jnp.maximum(
            jnp.dot(u, w1_ref[...], preferred_element_type=jnp.float32)
            + b1_ref[...], 0.0)
        z = jnp.maximum(
            jnp.dot(a, w2_ref[...], preferred_element_type=jnp.float32)
            + b2_ref[...], 0.0)
        if write_z:
            z_ref[...] = z

        ohb = oh_ref[...]
        gs = lax.dot_general(ohb, z, (((0,), (0,)), ((), ())),
                             preferred_element_type=jnp.float32)
        cn = lax.dot_general(ohb, jnp.ones((BLK, DIM), jnp.float32),
                             (((0,), (0,)), ((), ())),
                             preferred_element_type=jnp.float32)
        s0 = jnp.sum(z, axis=0, keepdims=True)
        s1 = jnp.sum(z * z, axis=0, keepdims=True)
        st = jnp.concatenate(
            [s0, s1, jnp.zeros((6, DIM), jnp.float32)], axis=0)

        init = b == 0
        stats_ref[...] = jnp.where(init, st, stats_ref[...] + st)
        gsum_ref[...] = jnp.where(init, gs, gsum_ref[...] + gs)
        cnt_ref[...] = jnp.where(init, cn, cnt_ref[...] + cn)

    zspec = [pl.BlockSpec((BLK, DIM), lambda b: (b, 0))] if write_z else []
    zshape = [jax.ShapeDtypeStruct((N, DIM), jnp.float32)] if write_z else []
    out = pl.pallas_call(
        body,
        grid=(NB,),
        in_specs=[
            pl.BlockSpec((BLK, DIM), lambda b: (b, 0)),
            pl.BlockSpec((NC, BLK, DIM), lambda b: (0, b, 0)),
            pl.BlockSpec((BLK, G), lambda b: (b, 0)),
            pl.BlockSpec((DIM, DIM), lambda b: (0, 0)),
            pl.BlockSpec((1, DIM), lambda b: (0, 0)),
            pl.BlockSpec((DIM, DIM), lambda b: (0, 0)),
            pl.BlockSpec((1, DIM), lambda b: (0, 0)),
        ],
        out_specs=zspec + [
            pl.BlockSpec((8, DIM), lambda b: (0, 0)),
            pl.BlockSpec((G, DIM), lambda b: (0, 0)),
            pl.BlockSpec((G, DIM), lambda b: (0, 0)),
        ],
        out_shape=zshape + [
            jax.ShapeDtypeStruct((8, DIM), jnp.float32),
            jax.ShapeDtypeStruct((G, DIM), jnp.float32),
            jax.ShapeDtypeStruct((G, DIM), jnp.float32),
        ],
    )(h, p, oh, W1, b1, W2, b2)
    if not write_z:
        return (None,) + tuple(out)
    return out


def _tc_norm(z, stats, gamma, beta):
    def body(z_ref, st_ref, g_ref, be_ref, h_ref):
        mean = st_ref[0:1, :] * (1.0 / N)
        msq = st_ref[1:2, :] * (1.0 / N)
        var = msq - mean * mean
        rstd = lax.rsqrt(var + 1e-5)
        h_ref[...] = (z_ref[...] - mean) * (rstd * g_ref[...]) + be_ref[...]

    return pl.pallas_call(
        body,
        grid=(NB,),
        in_specs=[
            pl.BlockSpec((BLK, DIM), lambda b: (b, 0)),
            pl.BlockSpec((8, DIM), lambda b: (0, 0)),
            pl.BlockSpec((1, DIM), lambda b: (0, 0)),
            pl.BlockSpec((1, DIM), lambda b: (0, 0)),
        ],
        out_specs=pl.BlockSpec((BLK, DIM), lambda b: (b, 0)),
        out_shape=jax.ShapeDtypeStruct((N, DIM), jnp.float32),
    )(z, stats, gamma, beta)


def _tc_combine(gsums, cnt, statss, gammas, betas):
    def body(g0, g1, g2, cnt_ref, s0, s1, s2, ga0, ga1, ga2,
             be0, be1, be2, out_ref):
        gr = (g0, g1, g2)
        sr = (s0, s1, s2)
        gar = (ga0, ga1, ga2)
        ber = (be0, be1, be2)
        c = cnt_ref[...]
        for i in range(L):
            mean = sr[i][0:1, :] * (1.0 / N)
            msq = sr[i][1:2, :] * (1.0 / N)
            var = msq - mean * mean
            rstd = lax.rsqrt(var + 1e-5)
            pool = ((gr[i][...] - c * mean) * (rstd * gar[i][...])
                    + c * ber[i][...])
            out_ref[:, DIM * i:DIM * (i + 1)] = pool

    full = lambda shp: pl.BlockSpec(shp, lambda: tuple(0 for _ in shp))
    return pl.pallas_call(
        body,
        in_specs=[full((G, DIM))] * 3 + [full((G, DIM))] + [full((8, DIM))] * 3
                 + [full((1, DIM))] * 6,
        out_specs=full((G, L * DIM)),
        out_shape=jax.ShapeDtypeStruct((G, L * DIM), jnp.float32),
    )(*gsums, cnt, *statss, *gammas, *betas)


def kernel(x, edge_index, batch,
           W1_0, b1_0, W2_0, b2_0, gamma_0, beta_0,
           W1_1, b1_1, W2_1, b2_1, gamma_1, beta_1,
           W1_2, b1_2, W2_2, b2_2, gamma_2, beta_2):
    e4 = edge_index.reshape(2, NW, CH, K)
    oh = (batch[:, None] == jnp.arange(G, dtype=jnp.int32)[None, :])
    oh = oh.astype(jnp.float32)

    Ws = [(W1_0, b1_0, W2_0, b2_0, gamma_0, beta_0),
          (W1_1, b1_1, W2_1, b2_1, gamma_1, beta_1),
          (W1_2, b1_2, W2_2, b2_2, gamma_2, beta_2)]

    h = x
    gsums, statss, gammas, betas = [], [], [], []
    cnt = None
    for i in range(L):
        W1, b1, W2, b2, ga, be = Ws[i]
        b1r = b1.reshape(1, DIM)
        b2r = b2.reshape(1, DIM)
        gar = ga.reshape(1, DIM)
        ber = be.reshape(1, DIM)
        p = _sc_agg(h, e4)
        z, stats, gsum, cnt = _tc_mlp(h, p, oh, W1, b1r, W2, b2r,
                                      write_z=i < L - 1)
        gsums.append(gsum)
        statss.append(stats)
        gammas.append(gar)
        betas.append(ber)
        if i < L - 1:
            h = _tc_norm(z, stats, gar, ber)
    return _tc_combine(gsums, cnt, statss, gammas, betas)


# speedup vs baseline: 3.2101x; 1.0397x over previous
import functools

import jax
import jax.numpy as jnp
from jax import lax
from jax.experimental import pallas as pl
from jax.experimental.pallas import tpu as pltpu
from jax.experimental.pallas import tpu_sc as plsc

N = 10000
E = 320000
DIM = 128
G = 64
L = 3

NC = 2
NS = 16
NW = NC * NS
EPT = E // NW
K = 125
CPG = 8
CH = 80
NGRP = CH // CPG
PADN = 10240
RPT = PADN // NS
ZR = 128

BLK = 2000
NB = N // BLK


def _sc_agg(h, e4):
    mesh = plsc.VectorSubcoreMesh(core_axis_name="c", subcore_axis_name="s")

    @functools.partial(
        pl.kernel,
        mesh=mesh,
        out_type=jax.ShapeDtypeStruct((NC, PADN, DIM), jnp.float32),
        scratch_types=[
            pltpu.VMEM((2, CPG, K), jnp.int32),
            pltpu.VMEM((2, CPG, K), jnp.int32),
            pltpu.VMEM((ZR, DIM), jnp.float32),
            pltpu.VMEM((ZR, DIM), jnp.float32),
            pltpu.VMEM_SHARED((PADN, DIM), jnp.float32),
            pltpu.SemaphoreType.DMA,
            pltpu.SemaphoreType.DMA,
            pltpu.SemaphoreType.DMA,
        ],
    )
    def agg(h_hbm, e_hbm, out_hbm,
            idx_s, idx_d, rows0, rows1, acc, sem0, sem1, semi):
        c = lax.axis_index("c")
        s = lax.axis_index("s")
        wid = s * NC + c

        pltpu.async_copy(e_hbm.at[0, wid, pl.ds(0, CPG)], idx_s.at[0], semi)
        pltpu.async_copy(e_hbm.at[1, wid, pl.ds(0, CPG)], idx_d.at[0], semi)
        pltpu.async_copy(e_hbm.at[0, wid, pl.ds(CPG, CPG)], idx_s.at[1], semi)
        pltpu.async_copy(e_hbm.at[1, wid, pl.ds(CPG, CPG)], idx_d.at[1], semi)

        def zb(i, carry):
            for j in range(8):
                rows0[i, pl.ds(j * 16, 16)] = jnp.zeros((16,), jnp.float32)
            return carry
        lax.fori_loop(0, ZR, zb, 0)
        row0 = s * RPT

        def zacc(j, carry):
            pltpu.async_copy(rows0, acc.at[pl.ds(row0 + j * ZR, ZR)], sem0)
            return carry
        lax.fori_loop(0, RPT // ZR, zacc, 0)

        def zdrain(j, carry):
            pltpu.make_async_copy(rows0, acc.at[pl.ds(row0, ZR)],
                                  sem0).wait()
            return carry
        lax.fori_loop(0, RPT // ZR, zdrain, 0)
        plsc.subcore_barrier()

        rows = (rows0.at[pl.ds(0, K)], rows1.at[pl.ds(0, K)])
        sems = (sem0, sem1)

        pltpu.make_async_copy(e_hbm.at[0, wid, pl.ds(0, CPG)],
                              idx_s.at[0], semi).wait()
        pltpu.make_async_copy(e_hbm.at[1, wid, pl.ds(0, CPG)],
                              idx_d.at[0], semi).wait()
        pltpu.async_copy(h_hbm.at[idx_s.at[0, 0]], rows[0], sems[0])

        def group(g, carry):
            p = g % 2
            for b in range(CPG):
                cur = b % 2
                nxt = 1 - cur
                if b < CPG - 1:
                    pltpu.async_copy(h_hbm.at[idx_s.at[p, b + 1]],
                                     rows[nxt], sems[nxt])
                else:
                    @pl.when(g < NGRP - 1)
                    def _():
                        pltpu.make_async_copy(
                            e_hbm.at[0, wid, pl.ds(0, CPG)],
                            idx_s.at[0], semi).wait()
                        pltpu.make_async_copy(
                            e_hbm.at[1, wid, pl.ds(0, CPG)],
                            idx_d.at[0], semi).wait()
                        pltpu.async_copy(h_hbm.at[idx_s.at[1 - p, 0]],
                                         rows[nxt], sems[nxt])
                pltpu.make_async_copy(h_hbm.at[idx_s.at[p, b]],
                                      rows[cur], sems[cur]).wait()
                pltpu.sync_copy(rows[cur], acc.at[idx_d.at[p, b]], add=True)
            @pl.when(g < NGRP - 2)
            def _():
                pltpu.async_copy(
                    e_hbm.at[0, wid, pl.ds((g + 2) * CPG, CPG)],
                    idx_s.at[p], semi)
                pltpu.async_copy(
                    e_hbm.at[1, wid, pl.ds((g + 2) * CPG, CPG)],
                    idx_d.at[p], semi)
            return carry
        lax.fori_loop(0, NGRP, group, 0)

        plsc.subcore_barrier()
        pltpu.sync_copy(acc.at[pl.ds(row0, RPT)],
                        out_hbm.at[c, pl.ds(row0, RPT)])

    return agg(h, e4)


def _tc_mlp(h, p, oh, W1, b1, W2, b2, write_z=True):
    def body(h_ref, p_ref, oh_ref, w1_ref, b1_ref, w2_ref, b2_ref,
             *out_refs):
        if write_z:
            z_ref, stats_ref, gsum_ref, cnt_ref = out_refs
        else:
            stats_ref, gsum_ref, cnt_ref = out_refs
        b = pl.program_id(0)
        u = h_ref[...] + p_ref[0] + p_ref[1]
        a = jnp.maximum(
            jnp.dot(u, w1_ref[...], preferred_element_type=jnp.float32)
            + b1_ref[...], 0.0)
        z = jnp.maximum(
            jnp.dot(a, w2_ref[...], preferred_element_type=jnp.float32)
            + b2_ref[...], 0.0)
        if write_z:
            z_ref[...] = z

        ohb = oh_ref[...]
        gs = lax.dot_general(ohb, z, (((0,), (0,)), ((), ())),
                             preferred_element_type=jnp.float32)
        cn = lax.dot_general(ohb, jnp.ones((BLK, DIM), jnp.float32),
                             (((0,), (0,)), ((), ())),
                             preferred_element_type=jnp.float32)
        s0 = jnp.sum(z, axis=0, keepdims=True)
        s1 = jnp.sum(z * z, axis=0, keepdims=True)
        st = jnp.concatenate(
            [s0, s1, jnp.zeros((6, DIM), jnp.float32)], axis=0)

        init = b == 0
        stats_ref[...] = jnp.where(init, st, stats_ref[...] + st)
        gsum_ref[...] = jnp.where(init, gs, gsum_ref[...] + gs)
        cnt_ref[...] = jnp.where(init, cn, cnt_ref[...] + cn)

    zspec = [pl.BlockSpec((BLK, DIM), lambda b: (b, 0))] if write_z else []
    zshape = [jax.ShapeDtypeStruct((N, DIM), jnp.float32)] if write_z else []
    out = pl.pallas_call(
        body,
        grid=(NB,),
        in_specs=[
            pl.BlockSpec((BLK, DIM), lambda b: (b, 0)),
            pl.BlockSpec((NC, BLK, DIM), lambda b: (0, b, 0)),
            pl.BlockSpec((BLK, G), lambda b: (b, 0)),
            pl.BlockSpec((DIM, DIM), lambda b: (0, 0)),
            pl.BlockSpec((1, DIM), lambda b: (0, 0)),
            pl.BlockSpec((DIM, DIM), lambda b: (0, 0)),
            pl.BlockSpec((1, DIM), lambda b: (0, 0)),
        ],
        out_specs=zspec + [
            pl.BlockSpec((8, DIM), lambda b: (0, 0)),
            pl.BlockSpec((G, DIM), lambda b: (0, 0)),
            pl.BlockSpec((G, DIM), lambda b: (0, 0)),
        ],
        out_shape=zshape + [
            jax.ShapeDtypeStruct((8, DIM), jnp.float32),
            jax.ShapeDtypeStruct((G, DIM), jnp.float32),
            jax.ShapeDtypeStruct((G, DIM), jnp.float32),
        ],
    )(h, p, oh, W1, b1, W2, b2)
    if not write_z:
        return (None,) + tuple(out)
    return out


def _tc_norm(z, stats, gamma, beta):
    def body(z_ref, st_ref, g_ref, be_ref, h_ref):
        mean = st_ref[0:1, :] * (1.0 / N)
        msq = st_ref[1:2, :] * (1.0 / N)
        var = msq - mean * mean
        rstd = lax.rsqrt(var + 1e-5)
        h_ref[...] = (z_ref[...] - mean) * (rstd * g_ref[...]) + be_ref[...]

    return pl.pallas_call(
        body,
        grid=(NB,),
        in_specs=[
            pl.BlockSpec((BLK, DIM), lambda b: (b, 0)),
            pl.BlockSpec((8, DIM), lambda b: (0, 0)),
            pl.BlockSpec((1, DIM), lambda b: (0, 0)),
            pl.BlockSpec((1, DIM), lambda b: (0, 0)),
        ],
        out_specs=pl.BlockSpec((BLK, DIM), lambda b: (b, 0)),
        out_shape=jax.ShapeDtypeStruct((N, DIM), jnp.float32),
    )(z, stats, gamma, beta)


def _tc_combine(gsums, cnt, statss, gammas, betas):
    def body(g0, g1, g2, cnt_ref, s0, s1, s2, ga0, ga1, ga2,
             be0, be1, be2, out_ref):
        gr = (g0, g1, g2)
        sr = (s0, s1, s2)
        gar = (ga0, ga1, ga2)
        ber = (be0, be1, be2)
        c = cnt_ref[...]
        for i in range(L):
            mean = sr[i][0:1, :] * (1.0 / N)
            msq = sr[i][1:2, :] * (1.0 / N)
            var = msq - mean * mean
            rstd = lax.rsqrt(var + 1e-5)
            pool = ((gr[i][...] - c * mean) * (rstd * gar[i][...])
                    + c * ber[i][...])
            out_ref[:, DIM * i:DIM * (i + 1)] = pool

    full = lambda shp: pl.BlockSpec(shp, lambda: tuple(0 for _ in shp))
    return pl.pallas_call(
        body,
        in_specs=[full((G, DIM))] * 3 + [full((G, DIM))] + [full((8, DIM))] * 3
                 + [full((1, DIM))] * 6,
        out_specs=full((G, L * DIM)),
        out_shape=jax.ShapeDtypeStruct((G, L * DIM), jnp.float32),
    )(*gsums, cnt, *statss, *gammas, *betas)


def kernel(x, edge_index, batch,
           W1_0, b1_0, W2_0, b2_0, gamma_0, beta_0,
           W1_1, b1_1, W2_1, b2_1, gamma_1, beta_1,
           W1_2, b1_2, W2_2, b2_2, gamma_2, beta_2):
    e4 = edge_index.reshape(2, NW, CH, K)
    oh = (batch[:, None] == jnp.arange(G, dtype=jnp.int32)[None, :])
    oh = oh.astype(jnp.float32)

    Ws = [(W1_0, b1_0, W2_0, b2_0, gamma_0, beta_0),
          (W1_1, b1_1, W2_1, b2_1, gamma_1, beta_1),
          (W1_2, b1_2, W2_2, b2_2, gamma_2, beta_2)]

    h = x
    gsums, statss, gammas, betas = [], [], [], []
    cnt = None
    for i in range(L):
        W1, b1, W2, b2, ga, be = Ws[i]
        b1r = b1.reshape(1, DIM)
        b2r = b2.reshape(1, DIM)
        gar = ga.reshape(1, DIM)
        ber = be.reshape(1, DIM)
        p = _sc_agg(h, e4)
        z, stats, gsum, cnt = _tc_mlp(h, p, oh, W1, b1r, W2, b2r,
                                      write_z=i < L - 1)
        gsums.append(gsum)
        statss.append(stats)
        gammas.append(gar)
        betas.append(ber)
        if i < L - 1:
            h = _tc_norm(z, stats, gar, ber)
    return _tc_combine(gsums, cnt, statss, gammas, betas)


# speedup vs baseline: 3.2706x; 1.0188x over previous
import functools

import jax
import jax.numpy as jnp
from jax import lax
from jax.experimental import pallas as pl
from jax.experimental.pallas import tpu as pltpu
from jax.experimental.pallas import tpu_sc as plsc

N = 10000
E = 320000
DIM = 128
G = 64
L = 3

NC = 2
NS = 16
NW = NC * NS
EPT = E // NW
K = 125
CPG = 8
CH = 80
NGRP = CH // CPG
PADN = 10240
RPT = PADN // NS
ZR = 128

BLK = 5000
NB = N // BLK


def _sc_agg(h, e4):
    mesh = plsc.VectorSubcoreMesh(core_axis_name="c", subcore_axis_name="s")

    @functools.partial(
        pl.kernel,
        mesh=mesh,
        out_type=jax.ShapeDtypeStruct((NC, PADN, DIM), jnp.float32),
        scratch_types=[
            pltpu.VMEM((2, CPG, K), jnp.int32),
            pltpu.VMEM((2, CPG, K), jnp.int32),
            pltpu.VMEM((ZR, DIM), jnp.float32),
            pltpu.VMEM((ZR, DIM), jnp.float32),
            pltpu.VMEM_SHARED((PADN, DIM), jnp.float32),
            pltpu.SemaphoreType.DMA,
            pltpu.SemaphoreType.DMA,
            pltpu.SemaphoreType.DMA,
        ],
    )
    def agg(h_hbm, e_hbm, out_hbm,
            idx_s, idx_d, rows0, rows1, acc, sem0, sem1, semi):
        c = lax.axis_index("c")
        s = lax.axis_index("s")
        wid = s * NC + c

        pltpu.async_copy(e_hbm.at[0, wid, pl.ds(0, CPG)], idx_s.at[0], semi)
        pltpu.async_copy(e_hbm.at[1, wid, pl.ds(0, CPG)], idx_d.at[0], semi)
        pltpu.async_copy(e_hbm.at[0, wid, pl.ds(CPG, CPG)], idx_s.at[1], semi)
        pltpu.async_copy(e_hbm.at[1, wid, pl.ds(CPG, CPG)], idx_d.at[1], semi)

        def zb(i, carry):
            for j in range(8):
                rows0[i, pl.ds(j * 16, 16)] = jnp.zeros((16,), jnp.float32)
            return carry
        lax.fori_loop(0, ZR, zb, 0)
        row0 = s * RPT

        def zacc(j, carry):
            pltpu.async_copy(rows0, acc.at[pl.ds(row0 + j * ZR, ZR)], sem0)
            return carry
        lax.fori_loop(0, RPT // ZR, zacc, 0)

        def zdrain(j, carry):
            pltpu.make_async_copy(rows0, acc.at[pl.ds(row0, ZR)],
                                  sem0).wait()
            return carry
        lax.fori_loop(0, RPT // ZR, zdrain, 0)
        plsc.subcore_barrier()

        rows = (rows0.at[pl.ds(0, K)], rows1.at[pl.ds(0, K)])
        sems = (sem0, sem1)

        pltpu.make_async_copy(e_hbm.at[0, wid, pl.ds(0, CPG)],
                              idx_s.at[0], semi).wait()
        pltpu.make_async_copy(e_hbm.at[1, wid, pl.ds(0, CPG)],
                              idx_d.at[0], semi).wait()
        pltpu.async_copy(h_hbm.at[idx_s.at[0, 0]], rows[0], sems[0])

        def group(g, carry):
            p = g % 2
            for b in range(CPG):
                cur = b % 2
                nxt = 1 - cur
                if b < CPG - 1:
                    pltpu.async_copy(h_hbm.at[idx_s.at[p, b + 1]],
                                     rows[nxt], sems[nxt])
                else:
                    @pl.when(g < NGRP - 1)
                    def _():
                        pltpu.make_async_copy(
                            e_hbm.at[0, wid, pl.ds(0, CPG)],
                            idx_s.at[0], semi).wait()
                        pltpu.make_async_copy(
                            e_hbm.at[1, wid, pl.ds(0, CPG)],
                            idx_d.at[0], semi).wait()
                        pltpu.async_copy(h_hbm.at[idx_s.at[1 - p, 0]],
                                         rows[nxt], sems[nxt])
                pltpu.make_async_copy(h_hbm.at[idx_s.at[p, b]],
                                      rows[cur], sems[cur]).wait()
                pltpu.sync_copy(rows[cur], acc.at[idx_d.at[p, b]], add=True)
            @pl.when(g < NGRP - 2)
            def _():
                pltpu.async_copy(
                    e_hbm.at[0, wid, pl.ds((g + 2) * CPG, CPG)],
                    idx_s.at[p], semi)
                pltpu.async_copy(
                    e_hbm.at[1, wid, pl.ds((g + 2) * CPG, CPG)],
                    idx_d.at[p], semi)
            return carry
        lax.fori_loop(0, NGRP, group, 0)

        plsc.subcore_barrier()
        pltpu.sync_copy(acc.at[pl.ds(row0, RPT)],
                        out_hbm.at[c, pl.ds(row0, RPT)])

    return agg(h, e4)


def _tc_mlp(h, p, oh, W1, b1, W2, b2, write_z=True):
    def body(h_ref, p_ref, oh_ref, w1_ref, b1_ref, w2_ref, b2_ref,
             *out_refs):
        if write_z:
            z_ref, stats_ref, gsum_ref, cnt_ref = out_refs
        else:
            stats_ref, gsum_ref, cnt_ref = out_refs
        b = pl.program_id(0)
        u = h_ref[...] + p_ref[0] + p_ref[1]
        a = jnp.maximum(
            jnp.dot(u, w1_ref[...], preferred_element_type=jnp.float32)
            + b1_ref[...], 0.0)
        z = jnp.maximum(
            jnp.dot(a, w2_ref[...], preferred_element_type=jnp.float32)
            + b2_ref[...], 0.0)
        if write_z:
            z_ref[...] = z

        ohb = oh_ref[...]
        gs = lax.dot_general(ohb, z, (((0,), (0,)), ((), ())),
                             preferred_element_type=jnp.float32)
        cn = lax.dot_general(ohb, jnp.ones((BLK, DIM), jnp.float32),
                             (((0,), (0,)), ((), ())),
                             preferred_element_type=jnp.float32)
        s0 = jnp.sum(z, axis=0, keepdims=True)
        s1 = jnp.sum(z * z, axis=0, keepdims=True)
        st = jnp.concatenate(
            [s0, s1, jnp.zeros((6, DIM), jnp.float32)], axis=0)

        init = b == 0
        stats_ref[...] = jnp.where(init, st, stats_ref[...] + st)
        gsum_ref[...] = jnp.where(init, gs, gsum_ref[...] + gs)
        cnt_ref[...] = jnp.where(init, cn, cnt_ref[...] + cn)

    zspec = [pl.BlockSpec((BLK, DIM), lambda b: (b, 0))] if write_z else []
    zshape = [jax.ShapeDtypeStruct((N, DIM), jnp.float32)] if write_z else []
    out = pl.pallas_call(
        body,
        grid=(NB,),
        in_specs=[
            pl.BlockSpec((BLK, DIM), lambda b: (b, 0)),
            pl.BlockSpec((NC, BLK, DIM), lambda b: (0, b, 0)),
            pl.BlockSpec((BLK, G), lambda b: (b, 0)),
            pl.BlockSpec((DIM, DIM), lambda b: (0, 0)),
            pl.BlockSpec((1, DIM), lambda b: (0, 0)),
            pl.BlockSpec((DIM, DIM), lambda b: (0, 0)),
            pl.BlockSpec((1, DIM), lambda b: (0, 0)),
        ],
        out_specs=zspec + [
            pl.BlockSpec((8, DIM), lambda b: (0, 0)),
            pl.BlockSpec((G, DIM), lambda b: (0, 0)),
            pl.BlockSpec((G, DIM), lambda b: (0, 0)),
        ],
        out_shape=zshape + [
            jax.ShapeDtypeStruct((8, DIM), jnp.float32),
            jax.ShapeDtypeStruct((G, DIM), jnp.float32),
            jax.ShapeDtypeStruct((G, DIM), jnp.float32),
        ],
    )(h, p, oh, W1, b1, W2, b2)
    if not write_z:
        return (None,) + tuple(out)
    return out


def _tc_norm(z, stats, gamma, beta):
    def body(z_ref, st_ref, g_ref, be_ref, h_ref):
        mean = st_ref[0:1, :] * (1.0 / N)
        msq = st_ref[1:2, :] * (1.0 / N)
        var = msq - mean * mean
        rstd = lax.rsqrt(var + 1e-5)
        h_ref[...] = (z_ref[...] - mean) * (rstd * g_ref[...]) + be_ref[...]

    return pl.pallas_call(
        body,
        grid=(NB,),
        in_specs=[
            pl.BlockSpec((BLK, DIM), lambda b: (b, 0)),
            pl.BlockSpec((8, DIM), lambda b: (0, 0)),
            pl.BlockSpec((1, DIM), lambda b: (0, 0)),
            pl.BlockSpec((1, DIM), lambda b: (0, 0)),
        ],
        out_specs=pl.BlockSpec((BLK, DIM), lambda b: (b, 0)),
        out_shape=jax.ShapeDtypeStruct((N, DIM), jnp.float32),
    )(z, stats, gamma, beta)


def _tc_combine(gsums, cnt, statss, gammas, betas):
    def body(g0, g1, g2, cnt_ref, s0, s1, s2, ga0, ga1, ga2,
             be0, be1, be2, out_ref):
        gr = (g0, g1, g2)
        sr = (s0, s1, s2)
        gar = (ga0, ga1, ga2)
        ber = (be0, be1, be2)
        c = cnt_ref[...]
        for i in range(L):
            mean = sr[i][0:1, :] * (1.0 / N)
            msq = sr[i][1:2, :] * (1.0 / N)
            var = msq - mean * mean
            rstd = lax.rsqrt(var + 1e-5)
            pool = ((gr[i][...] - c * mean) * (rstd * gar[i][...])
                    + c * ber[i][...])
            out_ref[:, DIM * i:DIM * (i + 1)] = pool

    full = lambda shp: pl.BlockSpec(shp, lambda: tuple(0 for _ in shp))
    return pl.pallas_call(
        body,
        in_specs=[full((G, DIM))] * 3 + [full((G, DIM))] + [full((8, DIM))] * 3
                 + [full((1, DIM))] * 6,
        out_specs=full((G, L * DIM)),
        out_shape=jax.ShapeDtypeStruct((G, L * DIM), jnp.float32),
    )(*gsums, cnt, *statss, *gammas, *betas)


def kernel(x, edge_index, batch,
           W1_0, b1_0, W2_0, b2_0, gamma_0, beta_0,
           W1_1, b1_1, W2_1, b2_1, gamma_1, beta_1,
           W1_2, b1_2, W2_2, b2_2, gamma_2, beta_2):
    e4 = edge_index.reshape(2, NW, CH, K)
    oh = (batch[:, None] == jnp.arange(G, dtype=jnp.int32)[None, :])
    oh = oh.astype(jnp.float32)

    Ws = [(W1_0, b1_0, W2_0, b2_0, gamma_0, beta_0),
          (W1_1, b1_1, W2_1, b2_1, gamma_1, beta_1),
          (W1_2, b1_2, W2_2, b2_2, gamma_2, beta_2)]

    h = x
    gsums, statss, gammas, betas = [], [], [], []
    cnt = None
    for i in range(L):
        W1, b1, W2, b2, ga, be = Ws[i]
        b1r = b1.reshape(1, DIM)
        b2r = b2.reshape(1, DIM)
        gar = ga.reshape(1, DIM)
        ber = be.reshape(1, DIM)
        p = _sc_agg(h, e4)
        z, stats, gsum, cnt = _tc_mlp(h, p, oh, W1, b1r, W2, b2r,
                                      write_z=i < L - 1)
        gsums.append(gsum)
        statss.append(stats)
        gammas.append(gar)
        betas.append(ber)
        if i < L - 1:
            h = _tc_norm(z, stats, gar, ber)
    return _tc_combine(gsums, cnt, statss, gammas, betas)
